# baseline (speedup 1.0000x reference)
import functools

import jax
import jax.numpy as jnp
from jax import lax
from jax.experimental import pallas as pl
from jax.experimental.pallas import tpu as pltpu
from jax.experimental.pallas import tpu_sc as plsc

N_NODES = 10000
N_EDGES = 320000
D = 128

NC = 2
NS = 16
NW = NC * NS
CHUNK = 128
NCHUNK = 80
NCHUNK_C0 = 160
NCHUNK_C1 = 0
E_PAD = NW * NCHUNK * CHUNK
TOT_CHUNKS = NW * NCHUNK
N_PAD = 10240
ROWS_PER_SUB = N_PAD // NS
DEGW = 128

_mesh = plsc.VectorSubcoreMesh(core_axis_name="c", subcore_axis_name="s")


@functools.partial(
    pl.kernel,
    out_type=jax.ShapeDtypeStruct((NC, N_PAD, DEGW), jnp.float32),
    mesh=_mesh,
    scratch_types=[
        pltpu.VMEM((CHUNK,), jnp.int32),
        pltpu.VMEM((CHUNK, DEGW), jnp.float32),
        pltpu.VMEM_SHARED((N_PAD, DEGW), jnp.float32),
    ],
)
def _sc_degree(dst_hbm, zeros_hbm, ones_hbm, out_hbm, di_v, ones_v, acc_sh):
    c = lax.axis_index("c")
    s = lax.axis_index("s")
    w = c * NS + s
    base = w * NCHUNK * CHUNK
    pltpu.sync_copy(zeros_hbm.at[pl.ds(s * ROWS_PER_SUB, ROWS_PER_SUB)],
                    acc_sh.at[pl.ds(s * ROWS_PER_SUB, ROWS_PER_SUB)])
    pltpu.sync_copy(ones_hbm, ones_v)
    plsc.subcore_barrier()

    def body(j, carry):
        pltpu.sync_copy(dst_hbm.at[pl.ds(base + j * CHUNK, CHUNK)], di_v)
        pltpu.sync_copy(ones_v, acc_sh.at[di_v], add=True)
        return carry

    lax.fori_loop(0, NCHUNK, body, 0)
    plsc.subcore_barrier()
    pltpu.sync_copy(acc_sh.at[pl.ds(s * ROWS_PER_SUB, ROWS_PER_SUB)],
                    out_hbm.at[c, pl.ds(s * ROWS_PER_SUB, ROWS_PER_SUB)])


@functools.partial(
    pl.kernel,
    out_type=jax.ShapeDtypeStruct((NC, N_PAD, D), jnp.float32),
    mesh=_mesh,
    scratch_types=[
        pltpu.VMEM((CHUNK,), jnp.int32),
        pltpu.VMEM((CHUNK,), jnp.int32),
        pltpu.VMEM((CHUNK,), jnp.int32),
        pltpu.VMEM((CHUNK,), jnp.int32),
        pltpu.VMEM((CHUNK, D), jnp.float32),
        pltpu.VMEM((CHUNK, D), jnp.float32),
        pltpu.VMEM_SHARED((N_PAD, D), jnp.float32),
        pltpu.SemaphoreType.DMA,
        pltpu.SemaphoreType.DMA,
    ],
)
def _sc_scatter(h_hbm, src_hbm, dst_hbm, zeros_hbm, out_hbm,
                si0, di0, si1, di1, rows0, rows1, acc_sh, sem0, sem1):
    c = lax.axis_index("c")
    s = lax.axis_index("s")
    nchunk = jnp.where(c == 0, NCHUNK_C0, NCHUNK_C1)
    base = jnp.where(c == 0, s * NCHUNK_C0,
                     NS * NCHUNK_C0 + s * NCHUNK_C1) * CHUNK
    pltpu.sync_copy(zeros_hbm.at[pl.ds(s * ROWS_PER_SUB, ROWS_PER_SUB)],
                    acc_sh.at[pl.ds(s * ROWS_PER_SUB, ROWS_PER_SUB)])
    plsc.subcore_barrier()

    npair = nchunk // 2

    @pl.when(nchunk > 0)
    def _run():
        pltpu.sync_copy(src_hbm.at[pl.ds(base, CHUNK)], si0)
        pltpu.sync_copy(dst_hbm.at[pl.ds(base, CHUNK)], di0)
        pltpu.async_copy(h_hbm.at[si0], rows0, sem0)

        def body(p, carry):
            j1 = 2 * p + 1
            pltpu.sync_copy(src_hbm.at[pl.ds(base + j1 * CHUNK, CHUNK)], si1)
            pltpu.sync_copy(dst_hbm.at[pl.ds(base + j1 * CHUNK, CHUNK)], di1)
            pltpu.async_copy(h_hbm.at[si1], rows1, sem1)
            pltpu.make_async_copy(h_hbm.at[si0], rows0, sem0).wait()
            pltpu.sync_copy(rows0, acc_sh.at[di0], add=True)

            @pl.when(p < npair - 1)
            def _():
                pltpu.sync_copy(src_hbm.at[pl.ds(base + (j1 + 1) * CHUNK, CHUNK)], si0)
                pltpu.sync_copy(dst_hbm.at[pl.ds(base + (j1 + 1) * CHUNK, CHUNK)], di0)
                pltpu.async_copy(h_hbm.at[si0], rows0, sem0)

            pltpu.make_async_copy(h_hbm.at[si1], rows1, sem1).wait()
            pltpu.sync_copy(rows1, acc_sh.at[di1], add=True)
            return carry

        lax.fori_loop(0, npair, body, 0)
    plsc.subcore_barrier()
    pltpu.sync_copy(acc_sh.at[pl.ds(s * ROWS_PER_SUB, ROWS_PER_SUB)],
                    out_hbm.at[c, pl.ds(s * ROWS_PER_SUB, ROWS_PER_SUB)])


BLK = 512
GRID = N_PAD // BLK


def _dinv(d_ref):
    return lax.rsqrt(d_ref[0, :, 0:1] + d_ref[1, :, 0:1] + 1.0)


def _tc_h1_body(x_ref, w_ref, deg_ref, o_ref):
    h = jnp.dot(x_ref[...], w_ref[...], preferred_element_type=jnp.float32)
    o_ref[...] = h * _dinv(deg_ref)


def _tc_mid_body(acc_ref, h1s_ref, deg_ref, w_ref, b_ref, o_ref):
    dinv = _dinv(deg_ref)
    z = (acc_ref[0] + acc_ref[1] + h1s_ref[...]) * dinv + b_ref[...]
    r = jnp.maximum(z, 0.0)
    o_ref[...] = jnp.dot(r, w_ref[...], preferred_element_type=jnp.float32) * dinv


def _tc_out_body(acc_ref, h2s_ref, deg_ref, b_ref, o_ref):
    dinv = _dinv(deg_ref)
    o_ref[...] = (acc_ref[0] + acc_ref[1] + h2s_ref[...]) * dinv + b_ref[...]


_row_spec = pl.BlockSpec((BLK, D), lambda i: (i, 0))
_deg_spec = pl.BlockSpec((NC, BLK, DEGW), lambda i: (0, i, 0))
_acc_spec = pl.BlockSpec((NC, BLK, D), lambda i: (0, i, 0))
_w_spec = pl.BlockSpec((D, D), lambda i: (0, 0))
_b_spec = pl.BlockSpec((1, D), lambda i: (0, 0))
_row_out = jax.ShapeDtypeStruct((N_PAD, D), jnp.float32)

_tc_h1 = pl.pallas_call(
    _tc_h1_body, grid=(GRID,),
    in_specs=[_row_spec, _w_spec, _deg_spec],
    out_specs=_row_spec, out_shape=_row_out)

_tc_mid = pl.pallas_call(
    _tc_mid_body, grid=(GRID,),
    in_specs=[_acc_spec, _row_spec, _deg_spec, _w_spec, _b_spec],
    out_specs=_row_spec, out_shape=_row_out)

_tc_out = pl.pallas_call(
    _tc_out_body, grid=(GRID,),
    in_specs=[_acc_spec, _row_spec, _deg_spec, _b_spec],
    out_specs=_row_spec, out_shape=_row_out)


@jax.jit
def kernel(x, edge_index, W1, b1, W2, b2):
    src = edge_index[0].astype(jnp.int32)
    dst = edge_index[1].astype(jnp.int32)
    pad = E_PAD - N_EDGES
    src_p = jnp.concatenate([src, jnp.zeros((pad,), jnp.int32)])
    pad_rows = N_NODES + jnp.arange(pad, dtype=jnp.int32) % (N_PAD - N_NODES)
    dst_p = jnp.concatenate([dst, pad_rows])

    x_p = jnp.zeros((N_PAD, D), jnp.float32).at[:N_NODES].set(x)
    zeros_deg = jnp.zeros((N_PAD, DEGW), jnp.float32)
    zeros_acc = jnp.zeros((N_PAD, D), jnp.float32)
    ones_deg = jnp.ones((CHUNK, DEGW), jnp.float32)

    deg = _sc_degree(dst_p, zeros_deg, ones_deg)
    h1s = _tc_h1(x_p, W1, deg)
    acc1 = _sc_scatter(h1s, src_p, dst_p, zeros_acc)
    h2s = _tc_mid(acc1, h1s, deg, W2, b1.reshape(1, D))
    acc2 = _sc_scatter(h2s, src_p, dst_p, zeros_acc)
    out = _tc_out(acc2, h2s, deg, b2.reshape(1, D))
    return out[:N_NODES]


# speedup vs baseline: 1.3261x; 1.3261x over previous
import functools

import jax
import jax.numpy as jnp
from jax import lax
from jax.experimental import pallas as pl
from jax.experimental.pallas import tpu as pltpu
from jax.experimental.pallas import tpu_sc as plsc

N_NODES = 10000
N_EDGES = 320000
D = 128

NC = 2
NS = 16
NW = NC * NS
CHUNK = 128
NCHUNK = 80
NCHUNK_C0 = 136
NCHUNK_C1 = 24
E_PAD = NW * NCHUNK * CHUNK
TOT_CHUNKS = NW * NCHUNK
N_PAD = 10240
ROWS_PER_SUB = N_PAD // NS
DEGW = 128

_mesh = plsc.VectorSubcoreMesh(core_axis_name="c", subcore_axis_name="s")


@functools.partial(
    pl.kernel,
    out_type=jax.ShapeDtypeStruct((NC, N_PAD, DEGW), jnp.float32),
    mesh=_mesh,
    scratch_types=[
        pltpu.VMEM((CHUNK,), jnp.int32),
        pltpu.VMEM((CHUNK, DEGW), jnp.float32),
        pltpu.VMEM_SHARED((N_PAD, DEGW), jnp.float32),
    ],
)
def _sc_degree(dst_hbm, zeros_hbm, ones_hbm, out_hbm, di_v, ones_v, acc_sh):
    c = lax.axis_index("c")
    s = lax.axis_index("s")
    w = c * NS + s
    base = w * NCHUNK * CHUNK
    pltpu.sync_copy(zeros_hbm.at[pl.ds(s * ROWS_PER_SUB, ROWS_PER_SUB)],
                    acc_sh.at[pl.ds(s * ROWS_PER_SUB, ROWS_PER_SUB)])
    pltpu.sync_copy(ones_hbm, ones_v)
    plsc.subcore_barrier()

    def body(j, carry):
        pltpu.sync_copy(dst_hbm.at[pl.ds(base + j * CHUNK, CHUNK)], di_v)
        pltpu.sync_copy(ones_v, acc_sh.at[di_v], add=True)
        return carry

    lax.fori_loop(0, NCHUNK, body, 0)
    plsc.subcore_barrier()
    pltpu.sync_copy(acc_sh.at[pl.ds(s * ROWS_PER_SUB, ROWS_PER_SUB)],
                    out_hbm.at[c, pl.ds(s * ROWS_PER_SUB, ROWS_PER_SUB)])


@functools.partial(
    pl.kernel,
    out_type=jax.ShapeDtypeStruct((NC, N_PAD, D), jnp.float32),
    mesh=_mesh,
    scratch_types=[
        pltpu.VMEM((CHUNK,), jnp.int32),
        pltpu.VMEM((CHUNK,), jnp.int32),
        pltpu.VMEM((CHUNK,), jnp.int32),
        pltpu.VMEM((CHUNK,), jnp.int32),
        pltpu.VMEM((CHUNK, D), jnp.float32),
        pltpu.VMEM((CHUNK, D), jnp.float32),
        pltpu.VMEM_SHARED((N_PAD, D), jnp.float32),
        pltpu.SemaphoreType.DMA,
        pltpu.SemaphoreType.DMA,
    ],
)
def _sc_scatter(h_hbm, src_hbm, dst_hbm, zeros_hbm, out_hbm,
                si0, di0, si1, di1, rows0, rows1, acc_sh, sem0, sem1):
    c = lax.axis_index("c")
    s = lax.axis_index("s")
    nchunk = jnp.where(c == 0, NCHUNK_C0, NCHUNK_C1)
    base = jnp.where(c == 0, s * NCHUNK_C0,
                     NS * NCHUNK_C0 + s * NCHUNK_C1) * CHUNK
    pltpu.sync_copy(zeros_hbm.at[pl.ds(s * ROWS_PER_SUB, ROWS_PER_SUB)],
                    acc_sh.at[pl.ds(s * ROWS_PER_SUB, ROWS_PER_SUB)])
    plsc.subcore_barrier()

    npair = nchunk // 2

    @pl.when(nchunk > 0)
    def _run():
        pltpu.sync_copy(src_hbm.at[pl.ds(base, CHUNK)], si0)
        pltpu.sync_copy(dst_hbm.at[pl.ds(base, CHUNK)], di0)
        pltpu.async_copy(h_hbm.at[si0], rows0, sem0)

        def body(p, carry):
            j1 = 2 * p + 1
            pltpu.sync_copy(src_hbm.at[pl.ds(base + j1 * CHUNK, CHUNK)], si1)
            pltpu.sync_copy(dst_hbm.at[pl.ds(base + j1 * CHUNK, CHUNK)], di1)
            pltpu.async_copy(h_hbm.at[si1], rows1, sem1)
            pltpu.make_async_copy(h_hbm.at[si0], rows0, sem0).wait()
            pltpu.sync_copy(rows0, acc_sh.at[di0], add=True)

            @pl.when(p < npair - 1)
            def _():
                pltpu.sync_copy(src_hbm.at[pl.ds(base + (j1 + 1) * CHUNK, CHUNK)], si0)
                pltpu.sync_copy(dst_hbm.at[pl.ds(base + (j1 + 1) * CHUNK, CHUNK)], di0)
                pltpu.async_copy(h_hbm.at[si0], rows0, sem0)

            pltpu.make_async_copy(h_hbm.at[si1], rows1, sem1).wait()
            pltpu.sync_copy(rows1, acc_sh.at[di1], add=True)
            return carry

        lax.fori_loop(0, npair, body, 0)
    plsc.subcore_barrier()
    pltpu.sync_copy(acc_sh.at[pl.ds(s * ROWS_PER_SUB, ROWS_PER_SUB)],
                    out_hbm.at[c, pl.ds(s * ROWS_PER_SUB, ROWS_PER_SUB)])


BLK = 512
GRID = N_PAD // BLK


def _dinv(d_ref):
    return lax.rsqrt(d_ref[0, :, 0:1] + d_ref[1, :, 0:1] + 1.0)


def _tc_h1_body(x_ref, w_ref, deg_ref, o_ref):
    h = jnp.dot(x_ref[...], w_ref[...], preferred_element_type=jnp.float32)
    o_ref[...] = h * _dinv(deg_ref)


def _tc_mid_body(acc_ref, h1s_ref, deg_ref, w_ref, b_ref, o_ref):
    dinv = _dinv(deg_ref)
    z = (acc_ref[0] + acc_ref[1] + h1s_ref[...]) * dinv + b_ref[...]
    r = jnp.maximum(z, 0.0)
    o_ref[...] = jnp.dot(r, w_ref[...], preferred_element_type=jnp.float32) * dinv


def _tc_out_body(acc_ref, h2s_ref, deg_ref, b_ref, o_ref):
    dinv = _dinv(deg_ref)
    o_ref[...] = (acc_ref[0] + acc_ref[1] + h2s_ref[...]) * dinv + b_ref[...]


_row_spec = pl.BlockSpec((BLK, D), lambda i: (i, 0))
_deg_spec = pl.BlockSpec((NC, BLK, DEGW), lambda i: (0, i, 0))
_acc_spec = pl.BlockSpec((NC, BLK, D), lambda i: (0, i, 0))
_w_spec = pl.BlockSpec((D, D), lambda i: (0, 0))
_b_spec = pl.BlockSpec((1, D), lambda i: (0, 0))
_row_out = jax.ShapeDtypeStruct((N_PAD, D), jnp.float32)

_tc_h1 = pl.pallas_call(
    _tc_h1_body, grid=(GRID,),
    in_specs=[_row_spec, _w_spec, _deg_spec],
    out_specs=_row_spec, out_shape=_row_out)

_tc_mid = pl.pallas_call(
    _tc_mid_body, grid=(GRID,),
    in_specs=[_acc_spec, _row_spec, _deg_spec, _w_spec, _b_spec],
    out_specs=_row_spec, out_shape=_row_out)

_tc_out = pl.pallas_call(
    _tc_out_body, grid=(GRID,),
    in_specs=[_acc_spec, _row_spec, _deg_spec, _b_spec],
    out_specs=_row_spec, out_shape=_row_out)


@jax.jit
def kernel(x, edge_index, W1, b1, W2, b2):
    src = edge_index[0].astype(jnp.int32)
    dst = edge_index[1].astype(jnp.int32)
    pad = E_PAD - N_EDGES
    src_p = jnp.concatenate([src, jnp.zeros((pad,), jnp.int32)])
    pad_rows = N_NODES + jnp.arange(pad, dtype=jnp.int32) % (N_PAD - N_NODES)
    dst_p = jnp.concatenate([dst, pad_rows])

    x_p = jnp.zeros((N_PAD, D), jnp.float32).at[:N_NODES].set(x)
    zeros_deg = jnp.zeros((N_PAD, DEGW), jnp.float32)
    zeros_acc = jnp.zeros((N_PAD, D), jnp.float32)
    ones_deg = jnp.ones((CHUNK, DEGW), jnp.float32)

    deg = _sc_degree(dst_p, zeros_deg, ones_deg)
    h1s = _tc_h1(x_p, W1, deg)
    acc1 = _sc_scatter(h1s, src_p, dst_p, zeros_acc)
    h2s = _tc_mid(acc1, h1s, deg, W2, b1.reshape(1, D))
    acc2 = _sc_scatter(h2s, src_p, dst_p, zeros_acc)
    out = _tc_out(acc2, h2s, deg, b2.reshape(1, D))
    return out[:N_NODES]


# speedup vs baseline: 1.4386x; 1.0848x over previous
import functools

import jax
import jax.numpy as jnp
from jax import lax
from jax.experimental import pallas as pl
from jax.experimental.pallas import tpu as pltpu
from jax.experimental.pallas import tpu_sc as plsc

N_NODES = 10000
N_EDGES = 320000
D = 128

NC = 2
NS = 16
NW = NC * NS
CHUNK = 128
NCHUNK = 80
NCHUNK_C0 = 148
NCHUNK_C1 = 12
E_PAD = NW * NCHUNK * CHUNK
TOT_CHUNKS = NW * NCHUNK
N_PAD = 10240
ROWS_PER_SUB = N_PAD // NS
DEGW = 128

_mesh = plsc.VectorSubcoreMesh(core_axis_name="c", subcore_axis_name="s")


@functools.partial(
    pl.kernel,
    out_type=jax.ShapeDtypeStruct((NC, N_PAD, DEGW), jnp.float32),
    mesh=_mesh,
    scratch_types=[
        pltpu.VMEM((CHUNK,), jnp.int32),
        pltpu.VMEM((CHUNK, DEGW), jnp.float32),
        pltpu.VMEM_SHARED((N_PAD, DEGW), jnp.float32),
    ],
)
def _sc_degree(dst_hbm, zeros_hbm, ones_hbm, out_hbm, di_v, ones_v, acc_sh):
    c = lax.axis_index("c")
    s = lax.axis_index("s")
    w = c * NS + s
    base = w * NCHUNK * CHUNK
    pltpu.sync_copy(zeros_hbm.at[pl.ds(s * ROWS_PER_SUB, ROWS_PER_SUB)],
                    acc_sh.at[pl.ds(s * ROWS_PER_SUB, ROWS_PER_SUB)])
    pltpu.sync_copy(ones_hbm, ones_v)
    plsc.subcore_barrier()

    def body(j, carry):
        pltpu.sync_copy(dst_hbm.at[pl.ds(base + j * CHUNK, CHUNK)], di_v)
        pltpu.sync_copy(ones_v, acc_sh.at[di_v], add=True)
        return carry

    lax.fori_loop(0, NCHUNK, body, 0)
    plsc.subcore_barrier()
    pltpu.sync_copy(acc_sh.at[pl.ds(s * ROWS_PER_SUB, ROWS_PER_SUB)],
                    out_hbm.at[c, pl.ds(s * ROWS_PER_SUB, ROWS_PER_SUB)])


@functools.partial(
    pl.kernel,
    out_type=jax.ShapeDtypeStruct((NC, N_PAD, D), jnp.float32),
    mesh=_mesh,
    scratch_types=[
        pltpu.VMEM((CHUNK,), jnp.int32),
        pltpu.VMEM((CHUNK,), jnp.int32),
        pltpu.VMEM((CHUNK,), jnp.int32),
        pltpu.VMEM((CHUNK,), jnp.int32),
        pltpu.VMEM((CHUNK, D), jnp.float32),
        pltpu.VMEM((CHUNK, D), jnp.float32),
        pltpu.VMEM_SHARED((N_PAD, D), jnp.float32),
        pltpu.SemaphoreType.DMA,
        pltpu.SemaphoreType.DMA,
    ],
)
def _sc_scatter(h_hbm, src_hbm, dst_hbm, zeros_hbm, out_hbm,
                si0, di0, si1, di1, rows0, rows1, acc_sh, sem0, sem1):
    c = lax.axis_index("c")
    s = lax.axis_index("s")
    nchunk = jnp.where(c == 0, NCHUNK_C0, NCHUNK_C1)
    base = jnp.where(c == 0, s * NCHUNK_C0,
                     NS * NCHUNK_C0 + s * NCHUNK_C1) * CHUNK
    pltpu.sync_copy(zeros_hbm.at[pl.ds(s * ROWS_PER_SUB, ROWS_PER_SUB)],
                    acc_sh.at[pl.ds(s * ROWS_PER_SUB, ROWS_PER_SUB)])
    plsc.subcore_barrier()

    npair = nchunk // 2

    @pl.when(nchunk > 0)
    def _run():
        pltpu.sync_copy(src_hbm.at[pl.ds(base, CHUNK)], si0)
        pltpu.sync_copy(dst_hbm.at[pl.ds(base, CHUNK)], di0)
        pltpu.async_copy(h_hbm.at[si0], rows0, sem0)

        def body(p, carry):
            j1 = 2 * p + 1
            pltpu.sync_copy(src_hbm.at[pl.ds(base + j1 * CHUNK, CHUNK)], si1)
            pltpu.sync_copy(dst_hbm.at[pl.ds(base + j1 * CHUNK, CHUNK)], di1)
            pltpu.async_copy(h_hbm.at[si1], rows1, sem1)
            pltpu.make_async_copy(h_hbm.at[si0], rows0, sem0).wait()
            pltpu.sync_copy(rows0, acc_sh.at[di0], add=True)

            @pl.when(p < npair - 1)
            def _():
                pltpu.sync_copy(src_hbm.at[pl.ds(base + (j1 + 1) * CHUNK, CHUNK)], si0)
                pltpu.sync_copy(dst_hbm.at[pl.ds(base + (j1 + 1) * CHUNK, CHUNK)], di0)
                pltpu.async_copy(h_hbm.at[si0], rows0, sem0)

            pltpu.make_async_copy(h_hbm.at[si1], rows1, sem1).wait()
            pltpu.sync_copy(rows1, acc_sh.at[di1], add=True)
            return carry

        lax.fori_loop(0, npair, body, 0)
    plsc.subcore_barrier()
    pltpu.sync_copy(acc_sh.at[pl.ds(s * ROWS_PER_SUB, ROWS_PER_SUB)],
                    out_hbm.at[c, pl.ds(s * ROWS_PER_SUB, ROWS_PER_SUB)])


BLK = 512
GRID = N_PAD // BLK


def _dinv(d_ref):
    return lax.rsqrt(d_ref[0, :, 0:1] + d_ref[1, :, 0:1] + 1.0)


def _tc_h1_body(x_ref, w_ref, deg_ref, o_ref):
    h = jnp.dot(x_ref[...], w_ref[...], preferred_element_type=jnp.float32)
    o_ref[...] = h * _dinv(deg_ref)


def _tc_mid_body(acc_ref, h1s_ref, deg_ref, w_ref, b_ref, o_ref):
    dinv = _dinv(deg_ref)
    z = (acc_ref[0] + acc_ref[1] + h1s_ref[...]) * dinv + b_ref[...]
    r = jnp.maximum(z, 0.0)
    o_ref[...] = jnp.dot(r, w_ref[...], preferred_element_type=jnp.float32) * dinv


def _tc_out_body(acc_ref, h2s_ref, deg_ref, b_ref, o_ref):
    dinv = _dinv(deg_ref)
    o_ref[...] = (acc_ref[0] + acc_ref[1] + h2s_ref[...]) * dinv + b_ref[...]


_row_spec = pl.BlockSpec((BLK, D), lambda i: (i, 0))
_deg_spec = pl.BlockSpec((NC, BLK, DEGW), lambda i: (0, i, 0))
_acc_spec = pl.BlockSpec((NC, BLK, D), lambda i: (0, i, 0))
_w_spec = pl.BlockSpec((D, D), lambda i: (0, 0))
_b_spec = pl.BlockSpec((1, D), lambda i: (0, 0))
_row_out = jax.ShapeDtypeStruct((N_PAD, D), jnp.float32)

_tc_h1 = pl.pallas_call(
    _tc_h1_body, grid=(GRID,),
    in_specs=[_row_spec, _w_spec, _deg_spec],
    out_specs=_row_spec, out_shape=_row_out)

_tc_mid = pl.pallas_call(
    _tc_mid_body, grid=(GRID,),
    in_specs=[_acc_spec, _row_spec, _deg_spec, _w_spec, _b_spec],
    out_specs=_row_spec, out_shape=_row_out)

_tc_out = pl.pallas_call(
    _tc_out_body, grid=(GRID,),
    in_specs=[_acc_spec, _row_spec, _deg_spec, _b_spec],
    out_specs=_row_spec, out_shape=_row_out)


@jax.jit
def kernel(x, edge_index, W1, b1, W2, b2):
    src = edge_index[0].astype(jnp.int32)
    dst = edge_index[1].astype(jnp.int32)
    pad = E_PAD - N_EDGES
    src_p = jnp.concatenate([src, jnp.zeros((pad,), jnp.int32)])
    pad_rows = N_NODES + jnp.arange(pad, dtype=jnp.int32) % (N_PAD - N_NODES)
    dst_p = jnp.concatenate([dst, pad_rows])

    x_p = jnp.zeros((N_PAD, D), jnp.float32).at[:N_NODES].set(x)
    zeros_deg = jnp.zeros((N_PAD, DEGW), jnp.float32)
    zeros_acc = jnp.zeros((N_PAD, D), jnp.float32)
    ones_deg = jnp.ones((CHUNK, DEGW), jnp.float32)

    deg = _sc_degree(dst_p, zeros_deg, ones_deg)
    h1s = _tc_h1(x_p, W1, deg)
    acc1 = _sc_scatter(h1s, src_p, dst_p, zeros_acc)
    h2s = _tc_mid(acc1, h1s, deg, W2, b1.reshape(1, D))
    acc2 = _sc_scatter(h2s, src_p, dst_p, zeros_acc)
    out = _tc_out(acc2, h2s, deg, b2.reshape(1, D))
    return out[:N_NODES]


# speedup vs baseline: 1.4393x; 1.0005x over previous
import functools

import jax
import jax.numpy as jnp
from jax import lax
from jax.experimental import pallas as pl
from jax.experimental.pallas import tpu as pltpu
from jax.experimental.pallas import tpu_sc as plsc

N_NODES = 10000
N_EDGES = 320000
D = 128

NC = 2
NS = 16
NW = NC * NS
CHUNK = 128
NCHUNK = 80
NCHUNK_C0 = 154
NCHUNK_C1 = 6
E_PAD = NW * NCHUNK * CHUNK
TOT_CHUNKS = NW * NCHUNK
N_PAD = 10240
ROWS_PER_SUB = N_PAD // NS
DEGW = 128

_mesh = plsc.VectorSubcoreMesh(core_axis_name="c", subcore_axis_name="s")


@functools.partial(
    pl.kernel,
    out_type=jax.ShapeDtypeStruct((NC, N_PAD, DEGW), jnp.float32),
    mesh=_mesh,
    scratch_types=[
        pltpu.VMEM((CHUNK,), jnp.int32),
        pltpu.VMEM((CHUNK, DEGW), jnp.float32),
        pltpu.VMEM_SHARED((N_PAD, DEGW), jnp.float32),
    ],
)
def _sc_degree(dst_hbm, zeros_hbm, ones_hbm, out_hbm, di_v, ones_v, acc_sh):
    c = lax.axis_index("c")
    s = lax.axis_index("s")
    w = c * NS + s
    base = w * NCHUNK * CHUNK
    pltpu.sync_copy(zeros_hbm.at[pl.ds(s * ROWS_PER_SUB, ROWS_PER_SUB)],
                    acc_sh.at[pl.ds(s * ROWS_PER_SUB, ROWS_PER_SUB)])
    pltpu.sync_copy(ones_hbm, ones_v)
    plsc.subcore_barrier()

    def body(j, carry):
        pltpu.sync_copy(dst_hbm.at[pl.ds(base + j * CHUNK, CHUNK)], di_v)
        pltpu.sync_copy(ones_v, acc_sh.at[di_v], add=True)
        return carry

    lax.fori_loop(0, NCHUNK, body, 0)
    plsc.subcore_barrier()
    pltpu.sync_copy(acc_sh.at[pl.ds(s * ROWS_PER_SUB, ROWS_PER_SUB)],
                    out_hbm.at[c, pl.ds(s * ROWS_PER_SUB, ROWS_PER_SUB)])


@functools.partial(
    pl.kernel,
    out_type=jax.ShapeDtypeStruct((NC, N_PAD, D), jnp.float32),
    mesh=_mesh,
    scratch_types=[
        pltpu.VMEM((CHUNK,), jnp.int32),
        pltpu.VMEM((CHUNK,), jnp.int32),
        pltpu.VMEM((CHUNK,), jnp.int32),
        pltpu.VMEM((CHUNK,), jnp.int32),
        pltpu.VMEM((CHUNK, D), jnp.float32),
        pltpu.VMEM((CHUNK, D), jnp.float32),
        pltpu.VMEM_SHARED((N_PAD, D), jnp.float32),
        pltpu.SemaphoreType.DMA,
        pltpu.SemaphoreType.DMA,
    ],
)
def _sc_scatter(h_hbm, src_hbm, dst_hbm, zeros_hbm, out_hbm,
                si0, di0, si1, di1, rows0, rows1, acc_sh, sem0, sem1):
    c = lax.axis_index("c")
    s = lax.axis_index("s")
    nchunk = jnp.where(c == 0, NCHUNK_C0, NCHUNK_C1)
    base = jnp.where(c == 0, s * NCHUNK_C0,
                     NS * NCHUNK_C0 + s * NCHUNK_C1) * CHUNK
    pltpu.sync_copy(zeros_hbm.at[pl.ds(s * ROWS_PER_SUB, ROWS_PER_SUB)],
                    acc_sh.at[pl.ds(s * ROWS_PER_SUB, ROWS_PER_SUB)])
    plsc.subcore_barrier()

    npair = nchunk // 2

    @pl.when(nchunk > 0)
    def _run():
        pltpu.sync_copy(src_hbm.at[pl.ds(base, CHUNK)], si0)
        pltpu.sync_copy(dst_hbm.at[pl.ds(base, CHUNK)], di0)
        pltpu.async_copy(h_hbm.at[si0], rows0, sem0)

        def body(p, carry):
            j1 = 2 * p + 1
            pltpu.sync_copy(src_hbm.at[pl.ds(base + j1 * CHUNK, CHUNK)], si1)
            pltpu.sync_copy(dst_hbm.at[pl.ds(base + j1 * CHUNK, CHUNK)], di1)
            pltpu.async_copy(h_hbm.at[si1], rows1, sem1)
            pltpu.make_async_copy(h_hbm.at[si0], rows0, sem0).wait()
            pltpu.sync_copy(rows0, acc_sh.at[di0], add=True)

            @pl.when(p < npair - 1)
            def _():
                pltpu.sync_copy(src_hbm.at[pl.ds(base + (j1 + 1) * CHUNK, CHUNK)], si0)
                pltpu.sync_copy(dst_hbm.at[pl.ds(base + (j1 + 1) * CHUNK, CHUNK)], di0)
                pltpu.async_copy(h_hbm.at[si0], rows0, sem0)

            pltpu.make_async_copy(h_hbm.at[si1], rows1, sem1).wait()
            pltpu.sync_copy(rows1, acc_sh.at[di1], add=True)
            return carry

        lax.fori_loop(0, npair, body, 0)
    plsc.subcore_barrier()
    pltpu.sync_copy(acc_sh.at[pl.ds(s * ROWS_PER_SUB, ROWS_PER_SUB)],
                    out_hbm.at[c, pl.ds(s * ROWS_PER_SUB, ROWS_PER_SUB)])


BLK = 512
GRID = N_PAD // BLK


def _dinv(d_ref):
    return lax.rsqrt(d_ref[0, :, 0:1] + d_ref[1, :, 0:1] + 1.0)


def _tc_h1_body(x_ref, w_ref, deg_ref, o_ref):
    h = jnp.dot(x_ref[...], w_ref[...], preferred_element_type=jnp.float32)
    o_ref[...] = h * _dinv(deg_ref)


def _tc_mid_body(acc_ref, h1s_ref, deg_ref, w_ref, b_ref, o_ref):
    dinv = _dinv(deg_ref)
    z = (acc_ref[0] + acc_ref[1] + h1s_ref[...]) * dinv + b_ref[...]
    r = jnp.maximum(z, 0.0)
    o_ref[...] = jnp.dot(r, w_ref[...], preferred_element_type=jnp.float32) * dinv


def _tc_out_body(acc_ref, h2s_ref, deg_ref, b_ref, o_ref):
    dinv = _dinv(deg_ref)
    o_ref[...] = (acc_ref[0] + acc_ref[1] + h2s_ref[...]) * dinv + b_ref[...]


_row_spec = pl.BlockSpec((BLK, D), lambda i: (i, 0))
_deg_spec = pl.BlockSpec((NC, BLK, DEGW), lambda i: (0, i, 0))
_acc_spec = pl.BlockSpec((NC, BLK, D), lambda i: (0, i, 0))
_w_spec = pl.BlockSpec((D, D), lambda i: (0, 0))
_b_spec = pl.BlockSpec((1, D), lambda i: (0, 0))
_row_out = jax.ShapeDtypeStruct((N_PAD, D), jnp.float32)

_tc_h1 = pl.pallas_call(
    _tc_h1_body, grid=(GRID,),
    in_specs=[_row_spec, _w_spec, _deg_spec],
    out_specs=_row_spec, out_shape=_row_out)

_tc_mid = pl.pallas_call(
    _tc_mid_body, grid=(GRID,),
    in_specs=[_acc_spec, _row_spec, _deg_spec, _w_spec, _b_spec],
    out_specs=_row_spec, out_shape=_row_out)

_tc_out = pl.pallas_call(
    _tc_out_body, grid=(GRID,),
    in_specs=[_acc_spec, _row_spec, _deg_spec, _b_spec],
    out_specs=_row_spec, out_shape=_row_out)


@jax.jit
def kernel(x, edge_index, W1, b1, W2, b2):
    src = edge_index[0].astype(jnp.int32)
    dst = edge_index[1].astype(jnp.int32)
    pad = E_PAD - N_EDGES
    src_p = jnp.concatenate([src, jnp.zeros((pad,), jnp.int32)])
    pad_rows = N_NODES + jnp.arange(pad, dtype=jnp.int32) % (N_PAD - N_NODES)
    dst_p = jnp.concatenate([dst, pad_rows])

    x_p = jnp.zeros((N_PAD, D), jnp.float32).at[:N_NODES].set(x)
    zeros_deg = jnp.zeros((N_PAD, DEGW), jnp.float32)
    zeros_acc = jnp.zeros((N_PAD, D), jnp.float32)
    ones_deg = jnp.ones((CHUNK, DEGW), jnp.float32)

    deg = _sc_degree(dst_p, zeros_deg, ones_deg)
    h1s = _tc_h1(x_p, W1, deg)
    acc1 = _sc_scatter(h1s, src_p, dst_p, zeros_acc)
    h2s = _tc_mid(acc1, h1s, deg, W2, b1.reshape(1, D))
    acc2 = _sc_scatter(h2s, src_p, dst_p, zeros_acc)
    out = _tc_out(acc2, h2s, deg, b2.reshape(1, D))
    return out[:N_NODES]


# speedup vs baseline: 1.4925x; 1.0369x over previous
import functools

import jax
import jax.numpy as jnp
from jax import lax
from jax.experimental import pallas as pl
from jax.experimental.pallas import tpu as pltpu
from jax.experimental.pallas import tpu_sc as plsc

N_NODES = 10000
N_EDGES = 320000
D = 128

NC = 2
NS = 16
NW = NC * NS
CHUNK = 128
NCHUNK = 80
NCHUNK_C0 = 154
NCHUNK_C1 = 6
E_PAD = NW * NCHUNK * CHUNK
TOT_CHUNKS = NW * NCHUNK
N_PAD = 10240
ROWS_PER_SUB = N_PAD // NS
DEGW = 128

_mesh = plsc.VectorSubcoreMesh(core_axis_name="c", subcore_axis_name="s")


@functools.partial(
    pl.kernel,
    out_type=jax.ShapeDtypeStruct((NC, N_PAD, DEGW), jnp.float32),
    mesh=_mesh,
    scratch_types=[
        pltpu.VMEM((CHUNK,), jnp.int32),
        pltpu.VMEM((CHUNK,), jnp.int32),
        pltpu.VMEM((CHUNK, DEGW), jnp.float32),
        pltpu.VMEM_SHARED((N_PAD, DEGW), jnp.float32),
        pltpu.SemaphoreType.DMA,
        pltpu.SemaphoreType.DMA,
    ],
)
def _sc_degree(dst_hbm, zeros_hbm, ones_hbm, out_hbm, di0, di1, ones_v,
               acc_sh, sem0, sem1):
    c = lax.axis_index("c")
    s = lax.axis_index("s")
    w = c * NS + s
    base = w * NCHUNK * CHUNK
    pltpu.sync_copy(zeros_hbm.at[pl.ds(s * ROWS_PER_SUB, ROWS_PER_SUB)],
                    acc_sh.at[pl.ds(s * ROWS_PER_SUB, ROWS_PER_SUB)])
    pltpu.sync_copy(ones_hbm, ones_v)
    plsc.subcore_barrier()

    npair = NCHUNK // 2
    pltpu.sync_copy(dst_hbm.at[pl.ds(base, CHUNK)], di0)
    pltpu.async_copy(ones_v, acc_sh.at[di0], sem0, add=True)

    def body(p, carry):
        j1 = 2 * p + 1
        pltpu.sync_copy(dst_hbm.at[pl.ds(base + j1 * CHUNK, CHUNK)], di1)
        pltpu.async_copy(ones_v, acc_sh.at[di1], sem1, add=True)
        pltpu.make_async_copy(ones_v, acc_sh.at[di0], sem0).wait()

        @pl.when(p < npair - 1)
        def _():
            pltpu.sync_copy(dst_hbm.at[pl.ds(base + (j1 + 1) * CHUNK, CHUNK)], di0)
            pltpu.async_copy(ones_v, acc_sh.at[di0], sem0, add=True)

        pltpu.make_async_copy(ones_v, acc_sh.at[di1], sem1).wait()
        return carry

    lax.fori_loop(0, npair, body, 0)
    plsc.subcore_barrier()
    pltpu.sync_copy(acc_sh.at[pl.ds(s * ROWS_PER_SUB, ROWS_PER_SUB)],
                    out_hbm.at[c, pl.ds(s * ROWS_PER_SUB, ROWS_PER_SUB)])


@functools.partial(
    pl.kernel,
    out_type=jax.ShapeDtypeStruct((NC, N_PAD, D), jnp.float32),
    mesh=_mesh,
    scratch_types=[
        pltpu.VMEM((CHUNK,), jnp.int32),
        pltpu.VMEM((CHUNK,), jnp.int32),
        pltpu.VMEM((CHUNK,), jnp.int32),
        pltpu.VMEM((CHUNK,), jnp.int32),
        pltpu.VMEM((CHUNK, D), jnp.float32),
        pltpu.VMEM((CHUNK, D), jnp.float32),
        pltpu.VMEM_SHARED((N_PAD, D), jnp.float32),
        pltpu.SemaphoreType.DMA,
        pltpu.SemaphoreType.DMA,
    ],
)
def _sc_scatter(h_hbm, src_hbm, dst_hbm, zeros_hbm, out_hbm,
                si0, di0, si1, di1, rows0, rows1, acc_sh, sem0, sem1):
    c = lax.axis_index("c")
    s = lax.axis_index("s")
    nchunk = jnp.where(c == 0, NCHUNK_C0, NCHUNK_C1)
    base = jnp.where(c == 0, s * NCHUNK_C0,
                     NS * NCHUNK_C0 + s * NCHUNK_C1) * CHUNK
    pltpu.sync_copy(zeros_hbm.at[pl.ds(s * ROWS_PER_SUB, ROWS_PER_SUB)],
                    acc_sh.at[pl.ds(s * ROWS_PER_SUB, ROWS_PER_SUB)])
    plsc.subcore_barrier()

    npair = nchunk // 2

    @pl.when(nchunk > 0)
    def _run():
        pltpu.sync_copy(src_hbm.at[pl.ds(base, CHUNK)], si0)
        pltpu.sync_copy(dst_hbm.at[pl.ds(base, CHUNK)], di0)
        pltpu.async_copy(h_hbm.at[si0], rows0, sem0)

        def body(p, carry):
            j1 = 2 * p + 1
            pltpu.sync_copy(src_hbm.at[pl.ds(base + j1 * CHUNK, CHUNK)], si1)
            pltpu.sync_copy(dst_hbm.at[pl.ds(base + j1 * CHUNK, CHUNK)], di1)
            pltpu.async_copy(h_hbm.at[si1], rows1, sem1)
            pltpu.make_async_copy(h_hbm.at[si0], rows0, sem0).wait()
            pltpu.sync_copy(rows0, acc_sh.at[di0], add=True)

            @pl.when(p < npair - 1)
            def _():
                pltpu.sync_copy(src_hbm.at[pl.ds(base + (j1 + 1) * CHUNK, CHUNK)], si0)
                pltpu.sync_copy(dst_hbm.at[pl.ds(base + (j1 + 1) * CHUNK, CHUNK)], di0)
                pltpu.async_copy(h_hbm.at[si0], rows0, sem0)

            pltpu.make_async_copy(h_hbm.at[si1], rows1, sem1).wait()
            pltpu.sync_copy(rows1, acc_sh.at[di1], add=True)
            return carry

        lax.fori_loop(0, npair, body, 0)
    plsc.subcore_barrier()
    pltpu.sync_copy(acc_sh.at[pl.ds(s * ROWS_PER_SUB, ROWS_PER_SUB)],
                    out_hbm.at[c, pl.ds(s * ROWS_PER_SUB, ROWS_PER_SUB)])


BLK = 512
GRID = N_PAD // BLK


def _dinv(d_ref):
    return lax.rsqrt(d_ref[0, :, 0:1] + d_ref[1, :, 0:1] + 1.0)


def _tc_h1_body(x_ref, w_ref, deg_ref, o_ref):
    h = jnp.dot(x_ref[...], w_ref[...], preferred_element_type=jnp.float32)
    o_ref[...] = h * _dinv(deg_ref)


def _tc_mid_body(acc_ref, h1s_ref, deg_ref, w_ref, b_ref, o_ref):
    dinv = _dinv(deg_ref)
    z = (acc_ref[0] + acc_ref[1] + h1s_ref[...]) * dinv + b_ref[...]
    r = jnp.maximum(z, 0.0)
    o_ref[...] = jnp.dot(r, w_ref[...], preferred_element_type=jnp.float32) * dinv


def _tc_out_body(acc_ref, h2s_ref, deg_ref, b_ref, o_ref):
    dinv = _dinv(deg_ref)
    o_ref[...] = (acc_ref[0] + acc_ref[1] + h2s_ref[...]) * dinv + b_ref[...]


_row_spec = pl.BlockSpec((BLK, D), lambda i: (i, 0))
_deg_spec = pl.BlockSpec((NC, BLK, DEGW), lambda i: (0, i, 0))
_acc_spec = pl.BlockSpec((NC, BLK, D), lambda i: (0, i, 0))
_w_spec = pl.BlockSpec((D, D), lambda i: (0, 0))
_b_spec = pl.BlockSpec((1, D), lambda i: (0, 0))
_row_out = jax.ShapeDtypeStruct((N_PAD, D), jnp.float32)

_tc_h1 = pl.pallas_call(
    _tc_h1_body, grid=(GRID,),
    in_specs=[_row_spec, _w_spec, _deg_spec],
    out_specs=_row_spec, out_shape=_row_out)

_tc_mid = pl.pallas_call(
    _tc_mid_body, grid=(GRID,),
    in_specs=[_acc_spec, _row_spec, _deg_spec, _w_spec, _b_spec],
    out_specs=_row_spec, out_shape=_row_out)

_tc_out = pl.pallas_call(
    _tc_out_body, grid=(GRID,),
    in_specs=[_acc_spec, _row_spec, _deg_spec, _b_spec],
    out_specs=_row_spec, out_shape=_row_out)


@jax.jit
def kernel(x, edge_index, W1, b1, W2, b2):
    src = edge_index[0].astype(jnp.int32)
    dst = edge_index[1].astype(jnp.int32)
    pad = E_PAD - N_EDGES
    src_p = jnp.concatenate([src, jnp.zeros((pad,), jnp.int32)])
    pad_rows = N_NODES + jnp.arange(pad, dtype=jnp.int32) % (N_PAD - N_NODES)
    dst_p = jnp.concatenate([dst, pad_rows])

    x_p = jnp.zeros((N_PAD, D), jnp.float32).at[:N_NODES].set(x)
    zeros_deg = jnp.zeros((N_PAD, DEGW), jnp.float32)
    zeros_acc = jnp.zeros((N_PAD, D), jnp.float32)
    ones_deg = jnp.ones((CHUNK, DEGW), jnp.float32)

    deg = _sc_degree(dst_p, zeros_deg, ones_deg)
    h1s = _tc_h1(x_p, W1, deg)
    acc1 = _sc_scatter(h1s, src_p, dst_p, zeros_acc)
    h2s = _tc_mid(acc1, h1s, deg, W2, b1.reshape(1, D))
    acc2 = _sc_scatter(h2s, src_p, dst_p, zeros_acc)
    out = _tc_out(acc2, h2s, deg, b2.reshape(1, D))
    return out[:N_NODES]
